# baseline (speedup 1.0000x reference)
import functools

import jax
import jax.numpy as jnp
from jax import lax
from jax.experimental import pallas as pl
from jax.experimental.pallas import tpu as pltpu
from jax.experimental.pallas import tpu_sc as plsc

N = 10000
E = 160000
D = 256
DH = 128
N_PAD = 10240
E_PAD = 163840
NT = 16
ROWS_T = N_PAD // NT
IROWS = E_PAD // 128
IROWS_T1 = IROWS // (2 * NT)

_mesh = plsc.VectorSubcoreMesh(core_axis_name="c", subcore_axis_name="s")


@functools.partial(
    pl.kernel,
    mesh=_mesh,
    out_type=jax.ShapeDtypeStruct((2 * N_PAD,), jnp.float32),
    scratch_types=[
        pltpu.VMEM((IROWS_T1, 128), jnp.int32),
        pltpu.VMEM((128,), jnp.float32),
        pltpu.VMEM((ROWS_T,), jnp.float32),
        pltpu.VMEM_SHARED((N_PAD,), jnp.float32),
    ],
)
def _deg_call(dst_hbm, out_hbm, idx_v, ones_v, zeros_v, deg_sh):
    c = lax.axis_index("c")
    s = lax.axis_index("s")

    row0 = c * (NT * IROWS_T1) + s * IROWS_T1
    pltpu.sync_copy(dst_hbm.at[pl.ds(row0, IROWS_T1)], idx_v)

    for i in range(128 // 16):
        ones_v[pl.ds(i * 16, 16)] = jnp.full((16,), 1.0, jnp.float32)

    def zbody(i, carry):
        zeros_v[pl.ds(i * 16, 16)] = jnp.zeros((16,), jnp.float32)
        return carry
    lax.fori_loop(0, ROWS_T // 16, zbody, 0)

    pltpu.sync_copy(zeros_v, deg_sh.at[pl.ds(s * ROWS_T, ROWS_T)])
    plsc.subcore_barrier()

    def body(k, carry):
        pltpu.sync_copy(ones_v, deg_sh.at[idx_v.at[k]], add=True)
        return carry
    lax.fori_loop(0, IROWS_T1, body, 0)
    plsc.subcore_barrier()

    pltpu.sync_copy(deg_sh.at[pl.ds(s * ROWS_T, ROWS_T)],
                    out_hbm.at[pl.ds(c * N_PAD + s * ROWS_T, ROWS_T)])


NBK = 10
BLK = N_PAD // NBK
BSH = 10
TROWS = 24
GCH = 64
EPT = IROWS // NT
BUFE = E_PAD + NBK * 2048
PAT = 1920


def _cnt_body(s_ref, t_ref):
    sv = s_ref[...]
    lane = lax.broadcasted_iota(jnp.int32, (1, 1, 128), 2)
    t = jnp.zeros((1, 1, 128), jnp.float32)
    bkt = sv >> BSH
    for b in range(NBK):
        tot = jnp.sum(jnp.where(bkt == b, 1.0, 0.0))
        t = t + jnp.where(lane == b, tot, 0.0)
    t_ref[...] = t


def _cnt_call(src3):
    return pl.pallas_call(
        _cnt_body,
        grid=(IROWS // 64,),
        in_specs=[pl.BlockSpec((64, 128), lambda i: (i, 0))],
        out_specs=pl.BlockSpec((1, 1, 128), lambda i: (i, 0, 0)),
        out_shape=jax.ShapeDtypeStruct((TROWS, 1, 128), jnp.float32),
    )(src3)


def _pos_body(s_ref, d_ref, t_ref, pos_ref, pk_ref, carry):
    i = pl.program_id(0)

    @pl.when(i == 0)
    def _():
        tv = jnp.sum(t_ref[...][:IROWS // 64, 0, :], axis=0)
        base = 0
        for b in range(NBK):
            carry[b] = base
            tot = tv[b].astype(jnp.int32)
            base = base + ((tot + 2047) // 2048) * 2048

    sv = s_ref[...]
    dv = d_ref[...]
    bkt = sv >> BSH
    r_i = lax.broadcasted_iota(jnp.int32, (64, 64), 0)
    c_i = lax.broadcasted_iota(jnp.int32, (64, 64), 1)
    tri = jnp.where(r_i > c_i, 1.0, 0.0)
    l_i = lax.broadcasted_iota(jnp.int32, (128, 128), 0)
    j_i = lax.broadcasted_iota(jnp.int32, (128, 128), 1)
    tri128 = jnp.where(l_i <= j_i, 1.0, 0.0)
    pos = jnp.zeros((64, 128), jnp.int32)
    for b in range(NBK):
        mf = jnp.where(bkt == b, 1.0, 0.0)
        cs = jnp.dot(mf, tri128, preferred_element_type=jnp.float32)
        rowtot = cs[:, 127:128]
        rowpref = jnp.dot(tri, rowtot, preferred_element_type=jnp.float32)
        rank = (cs - mf + rowpref).astype(jnp.int32)
        pos = jnp.where(bkt == b, carry[b] + rank, pos)
        carry[b] = carry[b] + jnp.sum(mf).astype(jnp.int32)
    pos_ref[...] = pos
    pk_ref[...] = ((sv & (BLK - 1)) << 14) | dv


def _pos_call(src3, dst3, t):
    return pl.pallas_call(
        _pos_body,
        grid=(IROWS // 64,),
        in_specs=[
            pl.BlockSpec((64, 128), lambda i: (i, 0)),
            pl.BlockSpec((64, 128), lambda i: (i, 0)),
            pl.BlockSpec((TROWS, 1, 128), lambda i: (0, 0, 0)),
        ],
        out_specs=[
            pl.BlockSpec((64, 128), lambda i: (i, 0)),
            pl.BlockSpec((64, 128), lambda i: (i, 0)),
        ],
        out_shape=[
            jax.ShapeDtypeStruct((IROWS, 128), jnp.int32),
            jax.ShapeDtypeStruct((IROWS, 128), jnp.int32),
        ],
        scratch_shapes=[pltpu.SMEM((16,), jnp.int32)],
    )(src3, dst3, t)


@functools.partial(
    pl.kernel,
    mesh=_mesh,
    out_type=jax.ShapeDtypeStruct((2 * N_PAD, DH), jnp.float32),
    scratch_types=[
        pltpu.VMEM((4, 128), jnp.int32),
        pltpu.VMEM((4, 128), jnp.int32),
        pltpu.VMEM((TROWS, 1, 128), jnp.float32),
        pltpu.VMEM((PAT,), jnp.int32),
        pltpu.VMEM((2, GCH), jnp.int32),
        pltpu.VMEM((2, GCH), jnp.int32),
        pltpu.VMEM((2, GCH), jnp.int32),
        pltpu.VMEM((2, GCH, DH), jnp.float32),
        pltpu.VMEM_SHARED((N_PAD, DH), jnp.float32),
        pltpu.VMEM_SHARED((BLK, DH), jnp.float32),
        pltpu.VMEM_SHARED((BUFE,), jnp.int32),
    ] + [pltpu.SemaphoreType.DMA] * 10,
)
def _agg_call(y_hbm, pk_hbm, pos_hbm, t_hbm, out_hbm, pkr_v, posr_v, tcv,
              pat_v, pkc_v, fsrc_v, fdst_v, rows_v, acc_sh, tbl_sh, buf_sh,
              *sems):
    c = lax.axis_index("c")
    s = lax.axis_index("s")
    p1sem = sems[0:4]
    s1sem = sems[4:8]
    lsem = sems[8:10]
    gsem = sems[0:2]
    ssem = sems[2:4]

    r0 = s * ROWS_T
    pltpu.sync_copy(y_hbm.at[pl.ds(c * N_PAD + r0, ROWS_T)],
                    acc_sh.at[pl.ds(r0, ROWS_T)])

    pltpu.sync_copy(t_hbm.at[pl.ds(0, TROWS)], tcv)
    cnt16 = jnp.zeros((16,), jnp.float32)
    for r in range(IROWS // 64):
        cnt16 = cnt16 + tcv[r, 0, pl.ds(0, 16)]
    base = 0
    bases = []
    gaps = []
    for b in range(NBK):
        tot = cnt16[b].astype(jnp.int32)
        gap = ((tot + 2047) // 2048) * 2048
        bases.append(base)
        gaps.append(gap)
        base = base + gap

    safe = jnp.full((16,), N, jnp.int32) + s
    for j in range(PAT // 16):
        pat_v[pl.ds(j * 16, 16)] = safe
    fill0 = s * (BUFE // NT)
    for j in range(BUFE // NT // PAT):
        pltpu.sync_copy(pat_v, buf_sh.at[pl.ds(fill0 + j * PAT, PAT)])
    plsc.subcore_barrier()

    p0 = s * EPT

    def p1start(r, slot):
        pltpu.async_copy(pk_hbm.at[p0 + r], pkr_v.at[slot], p1sem[slot])
        pltpu.async_copy(pos_hbm.at[p0 + r], posr_v.at[slot], p1sem[slot])

    def p1wait(slot):
        pltpu.make_async_copy(pk_hbm.at[0], pkr_v.at[slot], p1sem[slot]).wait()
        pltpu.make_async_copy(pos_hbm.at[0], posr_v.at[slot],
                              p1sem[slot]).wait()

    def s1start(slot):
        pltpu.async_copy(pkr_v.at[slot], buf_sh.at[posr_v.at[slot]],
                         s1sem[slot])

    def s1wait(slot):
        pltpu.make_async_copy(pkr_v.at[slot], buf_sh.at[posr_v.at[0]],
                              s1sem[slot]).wait()

    p1start(0, 0)
    p1start(1, 1)

    def p1loop(g, carry):
        for b2 in range(4):
            r = g * 4 + b2

            @pl.when(r >= 2)
            def _():
                s1wait((b2 + 2) % 4)

            @pl.when(r + 2 < EPT)
            def _():
                p1start(r + 2, (b2 + 2) % 4)

            p1wait(b2)
            s1start(b2)
        return carry
    lax.fori_loop(0, EPT // 4, p1loop, 0)
    s1wait(2)
    s1wait(3)
    plsc.subcore_barrier()

    def lstart(base_b, j, slot):
        off = pl.multiple_of(base_b + (j * NT + s) * GCH, 8)
        pltpu.async_copy(buf_sh.at[pl.ds(off, GCH)],
                         pkc_v.at[slot], lsem[slot])

    def lwait(slot):
        pltpu.make_async_copy(buf_sh.at[pl.ds(0, GCH)], pkc_v.at[slot],
                              lsem[slot]).wait()

    def unpack(slot):
        for h in range(GCH // 16):
            pk = pkc_v[slot, pl.ds(h * 16, 16)]
            fsrc_v[slot, pl.ds(h * 16, 16)] = pk >> 14
            fdst_v[slot, pl.ds(h * 16, 16)] = pk & ((1 << 14) - 1)

    def gstart(slot):
        pltpu.async_copy(tbl_sh.at[fsrc_v.at[slot]], rows_v.at[slot],
                         gsem[slot])

    def gwait(slot):
        pltpu.make_async_copy(tbl_sh.at[fsrc_v.at[0]], rows_v.at[slot],
                              gsem[slot]).wait()

    def sstart(slot):
        pltpu.async_copy(rows_v.at[slot], acc_sh.at[fdst_v.at[slot]],
                         ssem[slot], add=True)

    def swait(slot):
        pltpu.make_async_copy(rows_v.at[slot], acc_sh.at[fdst_v.at[0]],
                              ssem[slot]).wait()

    tpt = BLK // NT
    for blk in range(NBK):
        pltpu.sync_copy(
            y_hbm.at[pl.ds(c * N_PAD + blk * BLK + s * tpt, tpt)],
            tbl_sh.at[pl.ds(s * tpt, tpt)])
        plsc.subcore_barrier()

        bb = bases[blk]
        nj = gaps[blk] // (NT * GCH)

        @pl.when(nj > 0)
        def _():
            lstart(bb, 0, 0)

        @pl.when(nj > 1)
        def _():
            lstart(bb, 1, 1)

        @pl.when(nj > 0)
        def _():
            lwait(0)
            unpack(0)
            gstart(0)

        def go(g, carry):
            for b2 in range(2):
                m = g * 2 + b2
                rs = b2
                rs1 = 1 - b2

                @pl.when(m >= 1)
                def _():
                    swait(rs1)

                @pl.when(m + 1 < nj)
                def _():
                    lwait(rs1)
                    unpack(rs1)
                    gstart(rs1)

                @pl.when(m + 2 < nj)
                def _():
                    lstart(bb, m + 2, rs)

                gwait(rs)
                sstart(rs)
            return carry
        lax.fori_loop(0, nj // 2, go, 0)

        @pl.when(nj > 0)
        def _():
            swait(1)
        plsc.subcore_barrier()

    pltpu.sync_copy(acc_sh.at[pl.ds(r0, ROWS_T)],
                    out_hbm.at[pl.ds(c * N_PAD + r0, ROWS_T)])


_RB = 512


def _mm_body(x_ref, w_ref, dga_ref, dgb_ref, y_ref):
    dinv = lax.rsqrt(dga_ref[...] + dgb_ref[...] + 1.0)
    acc = jnp.dot(x_ref[...], w_ref[...], preferred_element_type=jnp.float32)
    y_ref[...] = acc * dinv[:, None]


def _mm_call(x_pad, w, dga, dgb):
    nb = N_PAD // _RB
    return pl.pallas_call(
        _mm_body,
        grid=(nb, 2),
        in_specs=[
            pl.BlockSpec((_RB, D), lambda i, h: (i, 0)),
            pl.BlockSpec((D, DH), lambda i, h: (0, h)),
            pl.BlockSpec((_RB,), lambda i, h: (i,)),
            pl.BlockSpec((_RB,), lambda i, h: (i,)),
        ],
        out_specs=pl.BlockSpec((_RB, DH), lambda i, h: (h * nb + i, 0)),
        out_shape=jax.ShapeDtypeStruct((2 * N_PAD, DH), jnp.float32),
    )(x_pad, w, dga, dgb)


def _fin_body(a_ref, dga_ref, dgb_ref, b_ref, o_ref):
    dinv = lax.rsqrt(dga_ref[...] + dgb_ref[...] + 1.0)
    o_ref[...] = jnp.tanh(a_ref[0] * dinv[:, None] + b_ref[...][None, :])


def _fin_call(a3, dga, dgb, b):
    nb = N_PAD // _RB
    return pl.pallas_call(
        _fin_body,
        grid=(nb, 2),
        in_specs=[
            pl.BlockSpec((1, _RB, DH), lambda i, h: (h, i, 0)),
            pl.BlockSpec((_RB,), lambda i, h: (i,)),
            pl.BlockSpec((_RB,), lambda i, h: (i,)),
            pl.BlockSpec((DH,), lambda i, h: (h,)),
        ],
        out_specs=pl.BlockSpec((_RB, DH), lambda i, h: (i, h)),
        out_shape=jax.ShapeDtypeStruct((N, D), jnp.float32),
    )(a3, dga, dgb, b)


def kernel(x, edge_index, W, b):
    x = x.astype(jnp.float32)
    src = edge_index[0].astype(jnp.int32)
    dst = edge_index[1].astype(jnp.int32)

    npe = E_PAD - E
    pad_src = jnp.zeros((npe,), jnp.int32)
    pad_dst = N + (jnp.arange(npe, dtype=jnp.int32) % (N_PAD - N))
    src3 = jnp.concatenate([src, pad_src]).reshape(IROWS, 128)
    dst3 = jnp.concatenate([dst, pad_dst]).reshape(IROWS, 128)
    x_pad = jnp.pad(x, ((0, N_PAD - N), (0, 0)))

    deg2 = _deg_call(dst3)
    dga, dgb = deg2[:N_PAD], deg2[N_PAD:]
    tcnt = _cnt_call(src3)
    pos, pkd = _pos_call(src3, dst3, tcnt)
    y2 = _mm_call(x_pad, W, dga, dgb)
    a2 = _agg_call(y2, pkd, pos, tcnt)
    return _fin_call(a2.reshape(2, N_PAD, DH), dga, dgb, b)


# speedup vs baseline: 1.1896x; 1.1896x over previous
import functools

import jax
import jax.numpy as jnp
from jax import lax
from jax.experimental import pallas as pl
from jax.experimental.pallas import tpu as pltpu
from jax.experimental.pallas import tpu_sc as plsc

N = 10000
E = 160000
D = 256
DH = 128
N_PAD = 10240
E_PAD = 163840
NT = 16
ROWS_T = N_PAD // NT
IROWS = E_PAD // 128
IROWS_T1 = IROWS // (2 * NT)

_mesh = plsc.VectorSubcoreMesh(core_axis_name="c", subcore_axis_name="s")


@functools.partial(
    pl.kernel,
    mesh=_mesh,
    out_type=jax.ShapeDtypeStruct((2 * N_PAD,), jnp.float32),
    scratch_types=[
        pltpu.VMEM((IROWS_T1, 128), jnp.int32),
        pltpu.VMEM((128,), jnp.float32),
        pltpu.VMEM((ROWS_T,), jnp.float32),
        pltpu.VMEM_SHARED((N_PAD,), jnp.float32),
    ],
)
def _deg_call(dst_hbm, out_hbm, idx_v, ones_v, zeros_v, deg_sh):
    c = lax.axis_index("c")
    s = lax.axis_index("s")

    row0 = c * (NT * IROWS_T1) + s * IROWS_T1
    pltpu.sync_copy(dst_hbm.at[pl.ds(row0, IROWS_T1)], idx_v)

    for i in range(128 // 16):
        ones_v[pl.ds(i * 16, 16)] = jnp.full((16,), 1.0, jnp.float32)

    def zbody(i, carry):
        zeros_v[pl.ds(i * 16, 16)] = jnp.zeros((16,), jnp.float32)
        return carry
    lax.fori_loop(0, ROWS_T // 16, zbody, 0)

    pltpu.sync_copy(zeros_v, deg_sh.at[pl.ds(s * ROWS_T, ROWS_T)])
    plsc.subcore_barrier()

    def body(k, carry):
        pltpu.sync_copy(ones_v, deg_sh.at[idx_v.at[k]], add=True)
        return carry
    lax.fori_loop(0, IROWS_T1, body, 0)
    plsc.subcore_barrier()

    pltpu.sync_copy(deg_sh.at[pl.ds(s * ROWS_T, ROWS_T)],
                    out_hbm.at[pl.ds(c * N_PAD + s * ROWS_T, ROWS_T)])


NBK = 10
BLK = N_PAD // NBK
BSH = 10
TROWS = 24
GCH = 64
EPT = IROWS // NT
BUFE = E_PAD + NBK * 2048
PAT = 1920
_PB = 128


def _pos_body(s_ref, d_ref, t_ref, pos_ref, pk_ref):
    sv = s_ref[...]
    dv = d_ref[...]
    bkt = sv >> BSH

    l_i = lax.broadcasted_iota(jnp.int32, (128, 128), 0)
    j_i = lax.broadcasted_iota(jnp.int32, (128, 128), 1)
    tri128 = jnp.where(l_i <= j_i, 1.0, 0.0)
    r_i = lax.broadcasted_iota(jnp.int32, (_PB, _PB), 0)
    c_i = lax.broadcasted_iota(jnp.int32, (_PB, _PB), 1)
    trib = jnp.where(r_i > c_i, 1.0, 0.0)
    lane = lax.broadcasted_iota(jnp.int32, (1, 1, 128), 2)

    tots = []
    for b in range(NBK):
        tots.append(jnp.sum(jnp.where(bkt == b, 1.0, 0.0)).astype(jnp.int32))
    base = 0
    t = jnp.zeros((1, 1, 128), jnp.float32)
    carry = []
    for b in range(NBK):
        t = t + jnp.where(lane == b, tots[b].astype(jnp.float32), 0.0)
        carry.append(base)
        base = base + ((tots[b] + 2047) // 2048) * 2048
    t_ref[0:1] = t

    for i in range(IROWS // _PB):
        svb = sv[i * _PB:(i + 1) * _PB, :]
        bkb = bkt[i * _PB:(i + 1) * _PB, :]
        pos = jnp.zeros((_PB, 128), jnp.int32)
        for b in range(NBK):
            mf = jnp.where(bkb == b, 1.0, 0.0)
            cs = jnp.dot(mf, tri128, preferred_element_type=jnp.float32)
            rowtot = cs[:, 127:128]
            rowpref = jnp.dot(trib, rowtot, preferred_element_type=jnp.float32)
            rank = (cs - mf + rowpref).astype(jnp.int32)
            pos = jnp.where(bkb == b, carry[b] + rank, pos)
            carry[b] = carry[b] + jnp.sum(mf).astype(jnp.int32)
        pos_ref[i * _PB:(i + 1) * _PB, :] = pos
    pk_ref[...] = ((sv & (BLK - 1)) << 14) | dv


def _pos_call(src3, dst3):
    return pl.pallas_call(
        _pos_body,
        out_shape=[
            jax.ShapeDtypeStruct((TROWS, 1, 128), jnp.float32),
            jax.ShapeDtypeStruct((IROWS, 128), jnp.int32),
            jax.ShapeDtypeStruct((IROWS, 128), jnp.int32),
        ],
    )(src3, dst3)


@functools.partial(
    pl.kernel,
    mesh=_mesh,
    out_type=jax.ShapeDtypeStruct((2 * N_PAD, DH), jnp.float32),
    scratch_types=[
        pltpu.VMEM((4, 128), jnp.int32),
        pltpu.VMEM((4, 128), jnp.int32),
        pltpu.VMEM((TROWS, 1, 128), jnp.float32),
        pltpu.VMEM((PAT,), jnp.int32),
        pltpu.VMEM((2, GCH), jnp.int32),
        pltpu.VMEM((2, GCH), jnp.int32),
        pltpu.VMEM((2, GCH), jnp.int32),
        pltpu.VMEM((2, GCH, DH), jnp.float32),
        pltpu.VMEM_SHARED((N_PAD, DH), jnp.float32),
        pltpu.VMEM_SHARED((BLK, DH), jnp.float32),
        pltpu.VMEM_SHARED((BUFE,), jnp.int32),
    ] + [pltpu.SemaphoreType.DMA] * 10,
)
def _agg_call(y_hbm, pk_hbm, pos_hbm, t_hbm, out_hbm, pkr_v, posr_v, tcv,
              pat_v, pkc_v, fsrc_v, fdst_v, rows_v, acc_sh, tbl_sh, buf_sh,
              *sems):
    c = lax.axis_index("c")
    s = lax.axis_index("s")
    p1sem = sems[0:4]
    s1sem = sems[4:8]
    lsem = sems[8:10]
    gsem = sems[0:2]
    ssem = sems[2:4]

    r0 = s * ROWS_T
    pltpu.sync_copy(y_hbm.at[pl.ds(c * N_PAD + r0, ROWS_T)],
                    acc_sh.at[pl.ds(r0, ROWS_T)])

    pltpu.sync_copy(t_hbm.at[pl.ds(0, TROWS)], tcv)
    cnt16 = tcv[0, 0, pl.ds(0, 16)]
    base = 0
    bases = []
    gaps = []
    for b in range(NBK):
        tot = cnt16[b].astype(jnp.int32)
        gap = ((tot + 2047) // 2048) * 2048
        bases.append(base)
        gaps.append(gap)
        base = base + gap

    safe = jnp.full((16,), N, jnp.int32) + s
    for j in range(PAT // 16):
        pat_v[pl.ds(j * 16, 16)] = safe
    fill0 = s * (BUFE // NT)
    for j in range(BUFE // NT // PAT):
        pltpu.sync_copy(pat_v, buf_sh.at[pl.ds(fill0 + j * PAT, PAT)])
    plsc.subcore_barrier()

    p0 = s * EPT

    def p1start(r, slot):
        pltpu.async_copy(pk_hbm.at[p0 + r], pkr_v.at[slot], p1sem[slot])
        pltpu.async_copy(pos_hbm.at[p0 + r], posr_v.at[slot], p1sem[slot])

    def p1wait(slot):
        pltpu.make_async_copy(pk_hbm.at[0], pkr_v.at[slot], p1sem[slot]).wait()
        pltpu.make_async_copy(pos_hbm.at[0], posr_v.at[slot],
                              p1sem[slot]).wait()

    def s1start(slot):
        pltpu.async_copy(pkr_v.at[slot], buf_sh.at[posr_v.at[slot]],
                         s1sem[slot])

    def s1wait(slot):
        pltpu.make_async_copy(pkr_v.at[slot], buf_sh.at[posr_v.at[0]],
                              s1sem[slot]).wait()

    p1start(0, 0)
    p1start(1, 1)

    def p1loop(g, carry):
        for b2 in range(4):
            r = g * 4 + b2

            @pl.when(r >= 2)
            def _():
                s1wait((b2 + 2) % 4)

            @pl.when(r + 2 < EPT)
            def _():
                p1start(r + 2, (b2 + 2) % 4)

            p1wait(b2)
            s1start(b2)
        return carry
    lax.fori_loop(0, EPT // 4, p1loop, 0)
    s1wait(2)
    s1wait(3)
    plsc.subcore_barrier()

    def lstart(base_b, j, slot):
        off = pl.multiple_of(base_b + (j * NT + s) * GCH, 8)
        pltpu.async_copy(buf_sh.at[pl.ds(off, GCH)],
                         pkc_v.at[slot], lsem[slot])

    def lwait(slot):
        pltpu.make_async_copy(buf_sh.at[pl.ds(0, GCH)], pkc_v.at[slot],
                              lsem[slot]).wait()

    def unpack(slot):
        for h in range(GCH // 16):
            pk = pkc_v[slot, pl.ds(h * 16, 16)]
            fsrc_v[slot, pl.ds(h * 16, 16)] = pk >> 14
            fdst_v[slot, pl.ds(h * 16, 16)] = pk & ((1 << 14) - 1)

    def gstart(slot):
        pltpu.async_copy(tbl_sh.at[fsrc_v.at[slot]], rows_v.at[slot],
                         gsem[slot])

    def gwait(slot):
        pltpu.make_async_copy(tbl_sh.at[fsrc_v.at[0]], rows_v.at[slot],
                              gsem[slot]).wait()

    def sstart(slot):
        pltpu.async_copy(rows_v.at[slot], acc_sh.at[fdst_v.at[slot]],
                         ssem[slot], add=True)

    def swait(slot):
        pltpu.make_async_copy(rows_v.at[slot], acc_sh.at[fdst_v.at[0]],
                              ssem[slot]).wait()

    tpt = BLK // NT
    for blk in range(NBK):
        pltpu.sync_copy(
            y_hbm.at[pl.ds(c * N_PAD + blk * BLK + s * tpt, tpt)],
            tbl_sh.at[pl.ds(s * tpt, tpt)])
        plsc.subcore_barrier()

        bb = bases[blk]
        nj = gaps[blk] // (NT * GCH)

        @pl.when(nj > 0)
        def _():
            lstart(bb, 0, 0)

        @pl.when(nj > 1)
        def _():
            lstart(bb, 1, 1)

        @pl.when(nj > 0)
        def _():
            lwait(0)
            unpack(0)
            gstart(0)

        def go(g, carry):
            for b2 in range(2):
                m = g * 2 + b2
                rs = b2
                rs1 = 1 - b2

                @pl.when(m >= 1)
                def _():
                    swait(rs1)

                @pl.when(m + 1 < nj)
                def _():
                    lwait(rs1)
                    unpack(rs1)
                    gstart(rs1)

                @pl.when(m + 2 < nj)
                def _():
                    lstart(bb, m + 2, rs)

                gwait(rs)
                sstart(rs)
            return carry
        lax.fori_loop(0, nj // 2, go, 0)

        @pl.when(nj > 0)
        def _():
            swait(1)
        plsc.subcore_barrier()

    pltpu.sync_copy(acc_sh.at[pl.ds(r0, ROWS_T)],
                    out_hbm.at[pl.ds(c * N_PAD + r0, ROWS_T)])


_RB = 512


def _mm_body(x_ref, w_ref, dga_ref, dgb_ref, y_ref):
    dinv = lax.rsqrt(dga_ref[...] + dgb_ref[...] + 1.0)
    acc = jnp.dot(x_ref[...], w_ref[...], preferred_element_type=jnp.float32)
    y_ref[...] = acc * dinv[:, None]


def _mm_call(x_pad, w, dga, dgb):
    nb = N_PAD // _RB
    return pl.pallas_call(
        _mm_body,
        grid=(nb, 2),
        in_specs=[
            pl.BlockSpec((_RB, D), lambda i, h: (i, 0)),
            pl.BlockSpec((D, DH), lambda i, h: (0, h)),
            pl.BlockSpec((_RB,), lambda i, h: (i,)),
            pl.BlockSpec((_RB,), lambda i, h: (i,)),
        ],
        out_specs=pl.BlockSpec((_RB, DH), lambda i, h: (h * nb + i, 0)),
        out_shape=jax.ShapeDtypeStruct((2 * N_PAD, DH), jnp.float32),
    )(x_pad, w, dga, dgb)


def _fin_body(a_ref, dga_ref, dgb_ref, b_ref, o_ref):
    dinv = lax.rsqrt(dga_ref[...] + dgb_ref[...] + 1.0)
    o_ref[...] = jnp.tanh(a_ref[0] * dinv[:, None] + b_ref[...][None, :])


def _fin_call(a3, dga, dgb, b):
    nb = N_PAD // _RB
    return pl.pallas_call(
        _fin_body,
        grid=(nb, 2),
        in_specs=[
            pl.BlockSpec((1, _RB, DH), lambda i, h: (h, i, 0)),
            pl.BlockSpec((_RB,), lambda i, h: (i,)),
            pl.BlockSpec((_RB,), lambda i, h: (i,)),
            pl.BlockSpec((DH,), lambda i, h: (h,)),
        ],
        out_specs=pl.BlockSpec((_RB, DH), lambda i, h: (i, h)),
        out_shape=jax.ShapeDtypeStruct((N, D), jnp.float32),
    )(a3, dga, dgb, b)


def kernel(x, edge_index, W, b):
    x = x.astype(jnp.float32)
    src = edge_index[0].astype(jnp.int32)
    dst = edge_index[1].astype(jnp.int32)

    npe = E_PAD - E
    pad_src = jnp.zeros((npe,), jnp.int32)
    pad_dst = N + (jnp.arange(npe, dtype=jnp.int32) % (N_PAD - N))
    src3 = jnp.concatenate([src, pad_src]).reshape(IROWS, 128)
    dst3 = jnp.concatenate([dst, pad_dst]).reshape(IROWS, 128)
    x_pad = jnp.pad(x, ((0, N_PAD - N), (0, 0)))

    deg2 = _deg_call(dst3)
    dga, dgb = deg2[:N_PAD], deg2[N_PAD:]
    tcnt, pos, pkd = _pos_call(src3, dst3)
    y2 = _mm_call(x_pad, W, dga, dgb)
    a2 = _agg_call(y2, pkd, pos, tcnt)
    return _fin_call(a2.reshape(2, N_PAD, DH), dga, dgb, b)
